# baseline (speedup 1.0000x reference)
import functools

import jax
import jax.numpy as jnp
from jax import lax
from jax.experimental import pallas as pl
from jax.experimental.pallas import tpu as pltpu
from jax.experimental.pallas import tpu_sc as plsc

D = 128
L = 16
NC, NS = 2, 16
NW = NC * NS
CHUNK = 160


def _make_sc_kernel(n_edges):
    per_w = n_edges // NW
    n_chunks = per_w // CHUNK
    assert n_edges == NW * CHUNK * n_chunks and n_chunks % 2 == 0
    mesh = plsc.VectorSubcoreMesh(
        core_axis_name="c", subcore_axis_name="s", num_cores=NC, num_subcores=NS
    )

    @functools.partial(
        pl.kernel,
        out_type=jax.ShapeDtypeStruct((n_edges,), jnp.float32),
        mesh=mesh,
        compiler_params=pltpu.CompilerParams(
            needs_layout_passes=False, use_tc_tiling_on_sc=False
        ),
        scratch_types=[
            pltpu.VMEM((CHUNK,), jnp.int32),
            pltpu.VMEM((CHUNK,), jnp.int32),
            pltpu.VMEM((CHUNK,), jnp.int32),
            pltpu.VMEM((CHUNK,), jnp.int32),
            pltpu.VMEM((CHUNK, D), jnp.float32),
            pltpu.VMEM((CHUNK, D), jnp.float32),
            pltpu.VMEM((CHUNK, D), jnp.float32),
            pltpu.VMEM((CHUNK, D), jnp.float32),
            pltpu.VMEM((CHUNK,), jnp.float32),
            pltpu.SemaphoreType.DMA,
            pltpu.SemaphoreType.DMA,
            pltpu.SemaphoreType.DMA,
            pltpu.SemaphoreType.DMA,
        ],
    )
    def sc_kernel(user_hbm, item_hbm, uidx_hbm, iidx_hbm, out_hbm,
                  uidx_a, iidx_a, uidx_b, iidx_b,
                  urows_a, irows_a, urows_b, irows_b,
                  out_v, usem_a, isem_a, usem_b, isem_b):
        wid = lax.axis_index("s") * NC + lax.axis_index("c")
        wbase = wid * per_w
        lane = lax.iota(jnp.int32, L)

        def issue(c, uidx_v, iidx_v, urows_v, irows_v, usem, isem):
            base = wbase + c * CHUNK
            pltpu.sync_copy(uidx_hbm.at[pl.ds(base, CHUNK)], uidx_v)
            pltpu.sync_copy(iidx_hbm.at[pl.ds(base, CHUNK)], iidx_v)
            pltpu.async_copy(user_hbm.at[uidx_v], urows_v, usem)
            pltpu.async_copy(item_hbm.at[iidx_v], irows_v, isem)

        def wait(uidx_v, iidx_v, urows_v, irows_v, usem, isem):
            pltpu.make_async_copy(user_hbm.at[uidx_v], urows_v, usem).wait()
            pltpu.make_async_copy(item_hbm.at[iidx_v], irows_v, isem).wait()

        def compute(c, urows_v, irows_v):
            def group_body(g, _):
                eidx = g * L + lane
                col = lane
                acc = plsc.load_gather(urows_v, [eidx, col]) * plsc.load_gather(
                    irows_v, [eidx, col])
                for d in range(1, D):
                    col = (lane + d) & (D - 1)
                    acc += plsc.load_gather(urows_v, [eidx, col]) * plsc.load_gather(
                        irows_v, [eidx, col])
                out_v[pl.ds(g * L, L)] = acc
                return 0

            lax.fori_loop(0, CHUNK // L, group_body, 0)
            pltpu.sync_copy(out_v, out_hbm.at[pl.ds(wbase + c * CHUNK, CHUNK)])

        bufs_a = (uidx_a, iidx_a, urows_a, irows_a, usem_a, isem_a)
        bufs_b = (uidx_b, iidx_b, urows_b, irows_b, usem_b, isem_b)

        issue(0, *bufs_a)

        def body(k, _):
            c0 = 2 * k
            issue(c0 + 1, *bufs_b)
            wait(*bufs_a)
            compute(c0, urows_a, irows_a)

            @pl.when(c0 + 2 < n_chunks)
            def _():
                issue(c0 + 2, *bufs_a)

            wait(*bufs_b)
            compute(c0 + 1, urows_b, irows_b)
            return 0

        lax.fori_loop(0, n_chunks // 2, body, 0)

    return sc_kernel


@jax.jit
def kernel(user_emb, item_emb, edge_index):
    n_edges = edge_index.shape[1]
    step = NW * CHUNK
    n_chunks = -(-n_edges // step)
    n_chunks += n_chunks % 2
    n_pad = step * n_chunks
    uidx = jnp.pad(edge_index[0], (0, n_pad - n_edges))
    iidx = jnp.pad(edge_index[1], (0, n_pad - n_edges))
    sc = _make_sc_kernel(n_pad)
    return sc(user_emb, item_emb, uidx, iidx)[:n_edges]


# speedup vs baseline: 1.2678x; 1.2678x over previous
import functools

import jax
import jax.numpy as jnp
from jax import lax
from jax.experimental import pallas as pl
from jax.experimental.pallas import tpu as pltpu
from jax.experimental.pallas import tpu_sc as plsc

D = 128
L = 16
NC, NS = 2, 16
NW = NC * NS
CHUNK = 480


def _make_sc_kernel(n_edges):
    per_w = n_edges // NW
    n_chunks = per_w // CHUNK
    assert n_edges == NW * CHUNK * n_chunks
    mesh = plsc.VectorSubcoreMesh(
        core_axis_name="c", subcore_axis_name="s", num_cores=NC, num_subcores=NS
    )

    @functools.partial(
        pl.kernel,
        out_type=jax.ShapeDtypeStruct((n_edges,), jnp.float32),
        mesh=mesh,
        compiler_params=pltpu.CompilerParams(
            needs_layout_passes=False, use_tc_tiling_on_sc=False
        ),
        scratch_types=[
            pltpu.VMEM((CHUNK,), jnp.int32),
            pltpu.VMEM((CHUNK,), jnp.int32),
            pltpu.VMEM((CHUNK, D), jnp.float32),
            pltpu.VMEM((CHUNK, D), jnp.float32),
            pltpu.VMEM((CHUNK,), jnp.float32),
            pltpu.SemaphoreType.DMA,
            pltpu.SemaphoreType.DMA,
        ],
    )
    def sc_kernel(user_hbm, item_hbm, uidx_hbm, iidx_hbm, out_hbm,
                  uidx_a, iidx_a, urows_a, irows_a,
                  out_v, usem_a, isem_a):
        wid = lax.axis_index("s") * NC + lax.axis_index("c")
        wbase = wid * per_w
        lane = lax.iota(jnp.int32, L)

        def issue(c, uidx_v, iidx_v, urows_v, irows_v, usem, isem):
            base = wbase + c * CHUNK
            pltpu.sync_copy(uidx_hbm.at[pl.ds(base, CHUNK)], uidx_v)
            pltpu.sync_copy(iidx_hbm.at[pl.ds(base, CHUNK)], iidx_v)
            pltpu.async_copy(user_hbm.at[uidx_v], urows_v, usem)
            pltpu.async_copy(item_hbm.at[iidx_v], irows_v, isem)

        def wait(uidx_v, iidx_v, urows_v, irows_v, usem, isem):
            pltpu.make_async_copy(user_hbm.at[uidx_v], urows_v, usem).wait()
            pltpu.make_async_copy(item_hbm.at[iidx_v], irows_v, isem).wait()

        def compute(c, urows_v, irows_v):
            def group_body(g, _):
                eidx = g * L + lane
                col = lane
                acc = plsc.load_gather(urows_v, [eidx, col]) * plsc.load_gather(
                    irows_v, [eidx, col])
                for d in range(1, D):
                    col = (lane + d) & (D - 1)
                    acc += plsc.load_gather(urows_v, [eidx, col]) * plsc.load_gather(
                        irows_v, [eidx, col])
                out_v[pl.ds(g * L, L)] = acc
                return 0

            lax.fori_loop(0, CHUNK // L, group_body, 0)
            pltpu.sync_copy(out_v, out_hbm.at[pl.ds(wbase + c * CHUNK, CHUNK)])

        bufs_a = (uidx_a, iidx_a, urows_a, irows_a, usem_a, isem_a)

        def body(c, _):
            issue(c, *bufs_a)
            wait(*bufs_a)
            compute(c, urows_a, irows_a)
            return 0

        lax.fori_loop(0, n_chunks, body, 0)

    return sc_kernel


@jax.jit
def kernel(user_emb, item_emb, edge_index):
    n_edges = edge_index.shape[1]
    step = NW * CHUNK
    n_chunks = -(-n_edges // step)
    n_pad = step * n_chunks
    uidx = jnp.pad(edge_index[0], (0, n_pad - n_edges))
    iidx = jnp.pad(edge_index[1], (0, n_pad - n_edges))
    sc = _make_sc_kernel(n_pad)
    return sc(user_emb, item_emb, uidx, iidx)[:n_edges]
